# baseline (speedup 1.0000x reference)
import functools

import jax
import jax.numpy as jnp
from jax import lax
from jax.experimental import pallas as pl
from jax.experimental.pallas import tpu as pltpu
from jax.experimental.pallas import tpu_sc as plsc

NUM_NODES = 1000000
NUM_PINS = 4000000

NC = 2
NS = 16
NW = NC * NS
LANES = 16

CHUNK = 16000
NUM_CHUNKS = NUM_PINS // CHUNK


def _pin_pos_body(nx_hbm, ny_hbm, idx_hbm, ox_hbm, oy_hbm, out_hbm,
                  idx_v, gx_v, gy_v, ox_v, oy_v, sem_x, sem_y):
    wid = lax.axis_index("s") * NC + lax.axis_index("c")
    n_mine = (NUM_CHUNKS - wid + NW - 1) // NW

    def chunk_body(t, _):
        chunk_id = wid + t * NW
        base = chunk_id * CHUNK
        pltpu.sync_copy(idx_hbm.at[pl.ds(base, CHUNK)], idx_v)
        cx = pltpu.async_copy(nx_hbm.at[idx_v], gx_v, sem_x)
        cy = pltpu.async_copy(ny_hbm.at[idx_v], gy_v, sem_y)
        pltpu.sync_copy(ox_hbm.at[pl.ds(base, CHUNK)], ox_v)
        pltpu.sync_copy(oy_hbm.at[pl.ds(base, CHUNK)], oy_v)
        cx.wait()
        cy.wait()

        def add_body(i, _):
            s = pl.ds(i * LANES, LANES)
            gx_v[s] = gx_v[s] + ox_v[s]
            gy_v[s] = gy_v[s] + oy_v[s]
            return 0

        lax.fori_loop(0, CHUNK // LANES, add_body, 0, unroll=4)
        pltpu.sync_copy(gx_v, out_hbm.at[pl.ds(base, CHUNK)])
        pltpu.sync_copy(gy_v, out_hbm.at[pl.ds(NUM_PINS + base, CHUNK)])
        return 0

    lax.fori_loop(0, n_mine, chunk_body, 0)


@jax.jit
def _pin_pos(node_x, node_y, idx, pin_offset_x, pin_offset_y):
    mesh = plsc.VectorSubcoreMesh(core_axis_name="c", subcore_axis_name="s",
                                  num_cores=NC, num_subcores=NS)
    return pl.kernel(
        _pin_pos_body,
        out_type=jax.ShapeDtypeStruct((2 * NUM_PINS,), jnp.float32),
        mesh=mesh,
        scratch_types=[
            pltpu.VMEM((CHUNK,), jnp.int32),
            pltpu.VMEM((CHUNK,), jnp.float32),
            pltpu.VMEM((CHUNK,), jnp.float32),
            pltpu.VMEM((CHUNK,), jnp.float32),
            pltpu.VMEM((CHUNK,), jnp.float32),
            pltpu.SemaphoreType.DMA,
            pltpu.SemaphoreType.DMA,
        ],
    )(node_x, node_y, idx, pin_offset_x, pin_offset_y)


def kernel(pos, pin_offset_x, pin_offset_y, pin2node_map,
           flat_node2pin_map, flat_node2pin_start_map):
    node_x = pos[:NUM_NODES]
    node_y = pos[NUM_NODES:]
    idx = pin2node_map.astype(jnp.int32)
    return _pin_pos(node_x, node_y, idx, pin_offset_x, pin_offset_y)


# speedup vs baseline: 1.9900x; 1.9900x over previous
import functools

import jax
import jax.numpy as jnp
from jax import lax
from jax.experimental import pallas as pl
from jax.experimental.pallas import tpu as pltpu
from jax.experimental.pallas import tpu_sc as plsc

NUM_NODES = 1000000
NUM_PHYSICAL_NODES = 900000
NUM_PINS = 4000000

NC = 2
NS = 16
LANES = 16

CHUNK = 20000
NUM_CHUNKS = NUM_PINS // CHUNK

STAGE = (NUM_PHYSICAL_NODES + 8 * NS - 1) // (8 * NS) * 8
TAB = STAGE * NS
BOUNCE = STAGE // 4


def _pin_pos_body(pos_hbm, idx_hbm, ox_hbm, oy_hbm, out_hbm,
                  tab_s, idx_v, g_v, o_v, sem_g):
    sid = lax.axis_index("s")
    cid = lax.axis_index("c")
    n_mine = (NUM_CHUNKS - sid + NS - 1) // NS

    def coord_plane(tab_base, off_hbm, out_base):
        s0 = sid * STAGE
        for k in range(STAGE // BOUNCE):
            b0 = s0 + k * BOUNCE
            pltpu.sync_copy(pos_hbm.at[pl.ds(tab_base + b0, BOUNCE)],
                            g_v.at[pl.ds(0, BOUNCE)])
            pltpu.sync_copy(g_v.at[pl.ds(0, BOUNCE)],
                            tab_s.at[pl.ds(b0, BOUNCE)])
        plsc.subcore_barrier()

        def chunk_body(t, _):
            chunk_id = sid + t * NS
            base = chunk_id * CHUNK
            pltpu.sync_copy(idx_hbm.at[pl.ds(base, CHUNK)], idx_v)
            cg = pltpu.async_copy(tab_s.at[idx_v], g_v, sem_g)
            pltpu.sync_copy(off_hbm.at[pl.ds(base, CHUNK)], o_v)
            cg.wait()

            def add_body(i, _):
                s = pl.ds(i * LANES, LANES)
                g_v[s] = g_v[s] + o_v[s]
                return 0

            lax.fori_loop(0, CHUNK // LANES, add_body, 0, unroll=4)
            pltpu.sync_copy(g_v, out_hbm.at[pl.ds(out_base + base, CHUNK)])
            return 0

        lax.fori_loop(0, n_mine, chunk_body, 0)

    @pl.when(cid == 0)
    def _():
        coord_plane(0, ox_hbm, 0)

    @pl.when(cid == 1)
    def _():
        coord_plane(NUM_NODES, oy_hbm, NUM_PINS)


@jax.jit
def _pin_pos(pos, idx, pin_offset_x, pin_offset_y):
    mesh = plsc.VectorSubcoreMesh(core_axis_name="c", subcore_axis_name="s",
                                  num_cores=NC, num_subcores=NS)
    return pl.kernel(
        _pin_pos_body,
        out_type=jax.ShapeDtypeStruct((2 * NUM_PINS,), jnp.float32),
        mesh=mesh,
        scratch_types=[
            pltpu.VMEM_SHARED((TAB,), jnp.float32),
            pltpu.VMEM((CHUNK,), jnp.int32),
            pltpu.VMEM((CHUNK,), jnp.float32),
            pltpu.VMEM((CHUNK,), jnp.float32),
            pltpu.SemaphoreType.DMA,
        ],
    )(pos, idx, pin_offset_x, pin_offset_y)


def kernel(pos, pin_offset_x, pin_offset_y, pin2node_map,
           flat_node2pin_map, flat_node2pin_start_map):
    idx = pin2node_map.astype(jnp.int32)
    return _pin_pos(pos, idx, pin_offset_x, pin_offset_y)


# speedup vs baseline: 2.7192x; 1.3664x over previous
import functools

import jax
import jax.numpy as jnp
from jax import lax
from jax.experimental import pallas as pl
from jax.experimental.pallas import tpu as pltpu
from jax.experimental.pallas import tpu_sc as plsc

NUM_NODES = 1000000
NUM_PHYSICAL_NODES = 900000
NUM_PINS = 4000000

NC = 2
NS = 16
LANES = 16

CHUNK = 10000
NUM_CHUNKS = NUM_PINS // CHUNK
NMAX = NUM_CHUNKS // NS

STAGE = (NUM_PHYSICAL_NODES + 8 * NS - 1) // (8 * NS) * 8
TAB = STAGE * NS
BOUNCE = STAGE // 8


def _pin_pos_body(pos_hbm, idx_hbm, ox_hbm, oy_hbm, out_hbm,
                  tab_s, idx0, idx1, g0, g1, o0, o1,
                  sg0, sg1, so0, so1, ss0, ss1):
    sid = lax.axis_index("s")
    cid = lax.axis_index("c")
    idx_v = (idx0, idx1)
    g_v = (g0, g1)
    o_v = (o0, o1)
    sem_g = (sg0, sg1)
    sem_o = (so0, so1)
    sem_s = (ss0, ss1)

    def coord_plane(tab_base, off_hbm, out_base):
        s0 = sid * STAGE
        for k in range(STAGE // BOUNCE):
            b0 = s0 + k * BOUNCE
            bb = g_v[k % 2]
            pltpu.sync_copy(pos_hbm.at[pl.ds(tab_base + b0, BOUNCE)],
                            bb.at[pl.ds(0, BOUNCE)])
            pltpu.sync_copy(bb.at[pl.ds(0, BOUNCE)],
                            tab_s.at[pl.ds(b0, BOUNCE)])
        plsc.subcore_barrier()

        def chunk_base(t):
            return (sid + t * NS) * CHUNK

        def fire(t):
            b = t % 2
            pltpu.sync_copy(idx_hbm.at[pl.ds(chunk_base(t), CHUNK)], idx_v[b])
            gd = pltpu.async_copy(tab_s.at[idx_v[b]], g_v[b], sem_g[b])
            od = pltpu.async_copy(off_hbm.at[pl.ds(chunk_base(t), CHUNK)],
                                  o_v[b], sem_o[b])
            return gd, od

        store_d = [None, None]
        pend = fire(0)
        for t in range(NMAX):
            b = t % 2
            nb = (t + 1) % 2
            if t + 1 < NMAX:
                if store_d[nb] is not None:
                    store_d[nb].wait()
                next_pend = fire(t + 1)
            pend[0].wait()
            pend[1].wait()

            def add_body(i, _):
                s = pl.ds(i * LANES, LANES)
                g_v[b][s] = g_v[b][s] + o_v[b][s]
                return 0

            lax.fori_loop(0, CHUNK // LANES, add_body, 0, unroll=8)
            store_d[b] = pltpu.async_copy(
                g_v[b], out_hbm.at[pl.ds(out_base + chunk_base(t), CHUNK)],
                sem_s[b])
            if t + 1 < NMAX:
                pend = next_pend
        store_d[0].wait()
        store_d[1].wait()

    @pl.when(cid == 0)
    def _():
        coord_plane(0, ox_hbm, 0)

    @pl.when(cid == 1)
    def _():
        coord_plane(NUM_NODES, oy_hbm, NUM_PINS)


@jax.jit
def _pin_pos(pos, idx, pin_offset_x, pin_offset_y):
    mesh = plsc.VectorSubcoreMesh(core_axis_name="c", subcore_axis_name="s",
                                  num_cores=NC, num_subcores=NS)
    return pl.kernel(
        _pin_pos_body,
        out_type=jax.ShapeDtypeStruct((2 * NUM_PINS,), jnp.float32),
        mesh=mesh,
        scratch_types=[
            pltpu.VMEM_SHARED((TAB,), jnp.float32),
            pltpu.VMEM((CHUNK,), jnp.int32),
            pltpu.VMEM((CHUNK,), jnp.int32),
            pltpu.VMEM((CHUNK,), jnp.float32),
            pltpu.VMEM((CHUNK,), jnp.float32),
            pltpu.VMEM((CHUNK,), jnp.float32),
            pltpu.VMEM((CHUNK,), jnp.float32),
            pltpu.SemaphoreType.DMA,
            pltpu.SemaphoreType.DMA,
            pltpu.SemaphoreType.DMA,
            pltpu.SemaphoreType.DMA,
            pltpu.SemaphoreType.DMA,
            pltpu.SemaphoreType.DMA,
        ],
    )(pos, idx, pin_offset_x, pin_offset_y)


def kernel(pos, pin_offset_x, pin_offset_y, pin2node_map,
           flat_node2pin_map, flat_node2pin_start_map):
    idx = pin2node_map.astype(jnp.int32)
    return _pin_pos(pos, idx, pin_offset_x, pin_offset_y)
